# initial kernel scaffold (unmeasured)
import jax
import jax.numpy as jnp
from jax import lax
from jax.experimental import pallas as pl
from jax.experimental.pallas import tpu as pltpu

N_Y = 4
V_SHARD = 8192


def kernel(ids, E):
    my_y = lax.axis_index("y")
    local = ids - my_y * V_SHARD
    in_range = (local >= 0) & (local < V_SHARD)
    rows = jnp.take(E, jnp.clip(local, 0, V_SHARD - 1), axis=0)
    partial = jnp.where(in_range[:, None], rows, jnp.float32(0.0))
    return _allreduce_y(partial)


def _allreduce_y(x):
    t, d = x.shape
    chunk = t // N_Y
    n_steps = 2 * (N_Y - 1)

    def body(x_hbm, out_ref, tmp_ref, send_sems, recv_sems, copy_sem):
        my_x = lax.axis_index("x")
        my_y = lax.axis_index("y")
        my_z = lax.axis_index("z")
        left = (my_y - 1) % N_Y
        right = (my_y + 1) % N_Y

        cp = pltpu.make_async_copy(x_hbm, out_ref, copy_sem)
        cp.start()

        barrier_sem = pltpu.get_barrier_semaphore()
        for nbr in (left, right):
            pl.semaphore_signal(
                barrier_sem,
                inc=1,
                device_id=(my_x, nbr, my_z),
                device_id_type=pl.DeviceIdType.MESH,
            )
        pl.semaphore_wait(barrier_sem, 2)
        cp.wait()

        for s in range(N_Y - 1):
            c = (my_y - s) % N_Y
            rdma = pltpu.make_async_remote_copy(
                src_ref=out_ref.at[pl.ds(c * chunk, chunk), :],
                dst_ref=tmp_ref.at[s],
                send_sem=send_sems.at[s],
                recv_sem=recv_sems.at[s],
                device_id=(my_x, right, my_z),
                device_id_type=pl.DeviceIdType.MESH,
            )
            rdma.start()
            rdma.wait()
            cr = (my_y - s - 1) % N_Y
            out_ref[pl.ds(cr * chunk, chunk), :] += tmp_ref[s]

        for g in range(N_Y - 1):
            s = (N_Y - 1) + g
            c = (my_y + 1 - g) % N_Y
            rdma = pltpu.make_async_remote_copy(
                src_ref=out_ref.at[pl.ds(c * chunk, chunk), :],
                dst_ref=out_ref.at[pl.ds(c * chunk, chunk), :],
                send_sem=send_sems.at[s],
                recv_sem=recv_sems.at[s],
                device_id=(my_x, right, my_z),
                device_id_type=pl.DeviceIdType.MESH,
            )
            rdma.start()
            rdma.wait()

    return pl.pallas_call(
        body,
        out_shape=jax.ShapeDtypeStruct((t, d), x.dtype),
        in_specs=[pl.BlockSpec(memory_space=pltpu.ANY)],
        out_specs=pl.BlockSpec(memory_space=pltpu.VMEM),
        scratch_shapes=[
            pltpu.VMEM((N_Y - 1, chunk, d), x.dtype),
            pltpu.SemaphoreType.DMA((n_steps,)),
            pltpu.SemaphoreType.DMA((n_steps,)),
            pltpu.SemaphoreType.DMA,
        ],
        compiler_params=pltpu.CompilerParams(collective_id=0),
    )(x)


# baseline (device time: 3366252 ns/iter reference)
import jax
import jax.numpy as jnp
from jax import lax
from jax.experimental import pallas as pl
from jax.experimental.pallas import tpu as pltpu

N_Y = 4
V_SHARD = 8192


def kernel(ids, E):
    my_y = lax.axis_index("y")
    local = ids - my_y * V_SHARD
    in_range = (local >= 0) & (local < V_SHARD)
    rows = jnp.take(E, jnp.clip(local, 0, V_SHARD - 1), axis=0)
    partial = jnp.where(in_range[:, None], rows, jnp.float32(0.0))
    return _allreduce_y(partial)


def _allreduce_y(x):
    t, d = x.shape
    chunk = t // N_Y
    n_steps = 2 * (N_Y - 1)

    def body(x_hbm, out_ref, tmp_ref, send_sems, recv_sems, copy_sem):
        my_x = lax.axis_index("x")
        my_y = lax.axis_index("y")
        my_z = lax.axis_index("z")
        left = (my_y - 1) % N_Y
        right = (my_y + 1) % N_Y

        cp = pltpu.make_async_copy(x_hbm, out_ref, copy_sem)
        cp.start()

        barrier_sem = pltpu.get_barrier_semaphore()
        for nbr in (left, right):
            pl.semaphore_signal(
                barrier_sem,
                inc=1,
                device_id=(my_x, nbr, my_z),
                device_id_type=pl.DeviceIdType.MESH,
            )
        pl.semaphore_wait(barrier_sem, 2)
        cp.wait()

        for s in range(N_Y - 1):
            c = (my_y - s) % N_Y
            rdma = pltpu.make_async_remote_copy(
                src_ref=out_ref.at[pl.ds(c * chunk, chunk), :],
                dst_ref=tmp_ref.at[s],
                send_sem=send_sems.at[s],
                recv_sem=recv_sems.at[s],
                device_id=(my_x, right, my_z),
                device_id_type=pl.DeviceIdType.MESH,
            )
            rdma.start()
            rdma.wait()
            cr = (my_y - s - 1) % N_Y
            out_ref[pl.ds(cr * chunk, chunk), :] += tmp_ref[s]

        for g in range(N_Y - 1):
            s = (N_Y - 1) + g
            c = (my_y + 1 - g) % N_Y
            rdma = pltpu.make_async_remote_copy(
                src_ref=out_ref.at[pl.ds(c * chunk, chunk), :],
                dst_ref=out_ref.at[pl.ds(c * chunk, chunk), :],
                send_sem=send_sems.at[s],
                recv_sem=recv_sems.at[s],
                device_id=(my_x, right, my_z),
                device_id_type=pl.DeviceIdType.MESH,
            )
            rdma.start()
            rdma.wait()

    return pl.pallas_call(
        body,
        out_shape=jax.ShapeDtypeStruct((t, d), x.dtype),
        in_specs=[pl.BlockSpec(memory_space=pl.ANY)],
        out_specs=pl.BlockSpec(memory_space=pltpu.VMEM),
        scratch_shapes=[
            pltpu.VMEM((N_Y - 1, chunk, d), x.dtype),
            pltpu.SemaphoreType.DMA((n_steps,)),
            pltpu.SemaphoreType.DMA((n_steps,)),
            pltpu.SemaphoreType.DMA,
        ],
        compiler_params=pltpu.CompilerParams(
            collective_id=0,
            vmem_limit_bytes=60 * 1024 * 1024,
        ),
    )(x)


# device time: 693735 ns/iter; 4.8524x vs baseline; 4.8524x over previous
import jax
import jax.numpy as jnp
from jax import lax
from jax.experimental import pallas as pl
from jax.experimental.pallas import tpu as pltpu

N_Y = 4
V_SHARD = 8192


def kernel(ids, E):
    my_y = lax.axis_index("y")
    local = ids - my_y * V_SHARD
    t = ids.shape[0]
    d = E.shape[1]
    chunk = t // N_Y
    n_steps = 2 * (N_Y - 1)

    def body(idx_ref, e_hbm, out_ref, tmp_ref, send_sems, recv_sems, gather_sem):
        my_x = lax.axis_index("x")
        my_y = lax.axis_index("y")
        my_z = lax.axis_index("z")
        left = (my_y - 1) % N_Y
        right = (my_y + 1) % N_Y

        barrier_sem = pltpu.get_barrier_semaphore()
        for nbr in (left, right):
            pl.semaphore_signal(
                barrier_sem,
                inc=1,
                device_id=(my_x, nbr, my_z),
                device_id_type=pl.DeviceIdType.MESH,
            )

        out_ref[...] = jnp.zeros((t, d), out_ref.dtype)

        def gather_one(tok, count):
            v = idx_ref[tok]
            valid = (v >= 0) & (v < V_SHARD)

            @pl.when(valid)
            def _():
                pltpu.make_async_copy(
                    e_hbm.at[pl.ds(v, 1), :],
                    out_ref.at[pl.ds(tok, 1), :],
                    gather_sem,
                ).start()

            return count + valid.astype(jnp.int32)

        count = lax.fori_loop(0, t, gather_one, jnp.int32(0))

        def drain_one(_, carry):
            pltpu.make_async_copy(
                e_hbm.at[pl.ds(0, 1), :],
                out_ref.at[pl.ds(0, 1), :],
                gather_sem,
            ).wait()
            return carry

        lax.fori_loop(0, count, drain_one, jnp.int32(0))

        pl.semaphore_wait(barrier_sem, 2)

        for s in range(N_Y - 1):
            c = (my_y - s) % N_Y
            rdma = pltpu.make_async_remote_copy(
                src_ref=out_ref.at[pl.ds(c * chunk, chunk), :],
                dst_ref=tmp_ref.at[s],
                send_sem=send_sems.at[s],
                recv_sem=recv_sems.at[s],
                device_id=(my_x, right, my_z),
                device_id_type=pl.DeviceIdType.MESH,
            )
            rdma.start()
            rdma.wait()
            cr = (my_y - s - 1) % N_Y
            out_ref[pl.ds(cr * chunk, chunk), :] += tmp_ref[s]

        for g in range(N_Y - 1):
            s = (N_Y - 1) + g
            c = (my_y + 1 - g) % N_Y
            rdma = pltpu.make_async_remote_copy(
                src_ref=out_ref.at[pl.ds(c * chunk, chunk), :],
                dst_ref=out_ref.at[pl.ds(c * chunk, chunk), :],
                send_sem=send_sems.at[s],
                recv_sem=recv_sems.at[s],
                device_id=(my_x, right, my_z),
                device_id_type=pl.DeviceIdType.MESH,
            )
            rdma.start()
            rdma.wait()

    return pl.pallas_call(
        body,
        out_shape=jax.ShapeDtypeStruct((t, d), E.dtype),
        in_specs=[
            pl.BlockSpec(memory_space=pltpu.SMEM),
            pl.BlockSpec(memory_space=pl.ANY),
        ],
        out_specs=pl.BlockSpec(memory_space=pltpu.VMEM),
        scratch_shapes=[
            pltpu.VMEM((N_Y - 1, chunk, d), E.dtype),
            pltpu.SemaphoreType.DMA((n_steps,)),
            pltpu.SemaphoreType.DMA((n_steps,)),
            pltpu.SemaphoreType.DMA,
        ],
        compiler_params=pltpu.CompilerParams(
            collective_id=0,
            vmem_limit_bytes=60 * 1024 * 1024,
        ),
    )(local, E)


# device time: 618982 ns/iter; 5.4384x vs baseline; 1.1208x over previous
import jax
import jax.numpy as jnp
from jax import lax
from jax.experimental import pallas as pl
from jax.experimental.pallas import tpu as pltpu

N_Y = 4
V_SHARD = 8192


def kernel(ids, E):
    my_y = lax.axis_index("y")
    local = ids - my_y * V_SHARD
    t = ids.shape[0]
    d = E.shape[1]
    chunk = t // N_Y
    dh = d // 2
    n_steps = 2 * (N_Y - 1)

    def body(idx_ref, e_hbm, out_ref, tmp_ref, send_sems, recv_sems, gather_sems):
        my_x = lax.axis_index("x")
        my_y = lax.axis_index("y")
        my_z = lax.axis_index("z")
        left = (my_y - 1) % N_Y
        right = (my_y + 1) % N_Y

        barrier_sem = pltpu.get_barrier_semaphore()
        for nbr in (left, right):
            pl.semaphore_signal(
                barrier_sem,
                inc=1,
                device_id=(my_x, nbr, my_z),
                device_id_type=pl.DeviceIdType.MESH,
            )

        out_ref[...] = jnp.zeros((t, d), out_ref.dtype)

        def scan_chunk(c, k):

            def gather_one(tok, count):
                v = idx_ref[tok]
                valid = (v >= 0) & (v < V_SHARD)

                @pl.when(valid)
                def _():
                    pltpu.make_async_copy(
                        e_hbm.at[pl.ds(v, 1), :],
                        out_ref.at[pl.ds(tok, 1), :],
                        gather_sems.at[k],
                    ).start()

                return count + valid.astype(jnp.int32)

            lo = c * chunk
            return lax.fori_loop(lo, lo + chunk, gather_one, jnp.int32(0))

        def drain_chunk(k, count):
            def drain_one(_, carry):
                pltpu.make_async_copy(
                    e_hbm.at[pl.ds(0, 1), :],
                    out_ref.at[pl.ds(0, 1), :],
                    gather_sems.at[k],
                ).wait()
                return carry

            lax.fori_loop(0, count, drain_one, jnp.int32(0))

        def rs_rdmas(s):
            c_cw = (my_y - s) % N_Y
            c_ccw = (my_y + s) % N_Y
            cw = pltpu.make_async_remote_copy(
                src_ref=out_ref.at[pl.ds(c_cw * chunk, chunk), pl.ds(0, dh)],
                dst_ref=tmp_ref.at[s, 0],
                send_sem=send_sems.at[s, 0],
                recv_sem=recv_sems.at[s, 0],
                device_id=(my_x, right, my_z),
                device_id_type=pl.DeviceIdType.MESH,
            )
            ccw = pltpu.make_async_remote_copy(
                src_ref=out_ref.at[pl.ds(c_ccw * chunk, chunk), pl.ds(dh, dh)],
                dst_ref=tmp_ref.at[s, 1],
                send_sem=send_sems.at[s, 1],
                recv_sem=recv_sems.at[s, 1],
                device_id=(my_x, left, my_z),
                device_id_type=pl.DeviceIdType.MESH,
            )
            return cw, ccw

        def rs_accumulate(s):
            r_cw = (my_y - s - 1) % N_Y
            r_ccw = (my_y + s + 1) % N_Y
            out_ref[pl.ds(r_cw * chunk, chunk), pl.ds(0, dh)] += tmp_ref[s, 0]
            out_ref[pl.ds(r_ccw * chunk, chunk), pl.ds(dh, dh)] += tmp_ref[s, 1]

        def ag_rdmas(g):
            s = (N_Y - 1) + g
            c_cw = (my_y + 1 - g) % N_Y
            c_ccw = (my_y - 1 + g) % N_Y
            cw = pltpu.make_async_remote_copy(
                src_ref=out_ref.at[pl.ds(c_cw * chunk, chunk), pl.ds(0, dh)],
                dst_ref=out_ref.at[pl.ds(c_cw * chunk, chunk), pl.ds(0, dh)],
                send_sem=send_sems.at[s, 0],
                recv_sem=recv_sems.at[s, 0],
                device_id=(my_x, right, my_z),
                device_id_type=pl.DeviceIdType.MESH,
            )
            ccw = pltpu.make_async_remote_copy(
                src_ref=out_ref.at[pl.ds(c_ccw * chunk, chunk), pl.ds(dh, dh)],
                dst_ref=out_ref.at[pl.ds(c_ccw * chunk, chunk), pl.ds(dh, dh)],
                send_sem=send_sems.at[s, 1],
                recv_sem=recv_sems.at[s, 1],
                device_id=(my_x, left, my_z),
                device_id_type=pl.DeviceIdType.MESH,
            )
            return cw, ccw

        cnt0 = scan_chunk(my_y, 0)
        drain_chunk(0, cnt0)

        pl.semaphore_wait(barrier_sem, 2)
        rs0_cw, rs0_ccw = rs_rdmas(0)
        rs0_cw.start()
        rs0_ccw.start()

        cnt1 = scan_chunk(left, 1)
        drain_chunk(1, cnt1)
        cnt2 = scan_chunk(right, 2)
        drain_chunk(2, cnt2)

        rs0_cw.wait()
        rs0_ccw.wait()
        rs_accumulate(0)

        rs1_cw, rs1_ccw = rs_rdmas(1)
        rs1_cw.start()
        rs1_ccw.start()

        cnt3 = scan_chunk((my_y + 2) % N_Y, 3)
        drain_chunk(3, cnt3)

        rs1_cw.wait()
        rs1_ccw.wait()
        rs_accumulate(1)

        rs2_cw, rs2_ccw = rs_rdmas(2)
        rs2_cw.start()
        rs2_ccw.start()
        rs2_cw.wait()
        rs2_ccw.wait()
        rs_accumulate(2)

        for g in range(N_Y - 1):
            ag_cw, ag_ccw = ag_rdmas(g)
            ag_cw.start()
            ag_ccw.start()
            ag_cw.wait()
            ag_ccw.wait()

    return pl.pallas_call(
        body,
        out_shape=jax.ShapeDtypeStruct((t, d), E.dtype),
        in_specs=[
            pl.BlockSpec(memory_space=pltpu.SMEM),
            pl.BlockSpec(memory_space=pl.ANY),
        ],
        out_specs=pl.BlockSpec(memory_space=pltpu.VMEM),
        scratch_shapes=[
            pltpu.VMEM((N_Y - 1, 2, chunk, dh), E.dtype),
            pltpu.SemaphoreType.DMA((n_steps, 2)),
            pltpu.SemaphoreType.DMA((n_steps, 2)),
            pltpu.SemaphoreType.DMA((N_Y,)),
        ],
        compiler_params=pltpu.CompilerParams(
            collective_id=0,
            vmem_limit_bytes=60 * 1024 * 1024,
        ),
    )(local, E)


# device time: 594377 ns/iter; 5.6635x vs baseline; 1.0414x over previous
import os

import jax
import jax.numpy as jnp

_SKIP_GATHER = bool(os.environ.get("SKIP_GATHER"))
_SKIP_RING = bool(os.environ.get("SKIP_RING"))
from jax import lax
from jax.experimental import pallas as pl
from jax.experimental.pallas import tpu as pltpu

N_Y = 4
V_SHARD = 8192


def kernel(ids, E):
    my_y = lax.axis_index("y")
    local = ids - my_y * V_SHARD
    t = ids.shape[0]
    d = E.shape[1]
    chunk = t // N_Y
    dh = d // 2
    n_steps = 2 * (N_Y - 1)

    def body(idx_ref, e_hbm, out_ref, tmp_ref, send_sems, recv_sems, gather_sems):
        my_x = lax.axis_index("x")
        my_y = lax.axis_index("y")
        my_z = lax.axis_index("z")
        left = (my_y - 1) % N_Y
        right = (my_y + 1) % N_Y

        if not _SKIP_RING:
            barrier_sem = pltpu.get_barrier_semaphore()
            for nbr in (left, right):
                pl.semaphore_signal(
                    barrier_sem,
                    inc=1,
                    device_id=(my_x, nbr, my_z),
                    device_id_type=pl.DeviceIdType.MESH,
                )

        out_ref[...] = jnp.zeros((t, d), out_ref.dtype)

        def scan_chunk(c, k):

            def gather_one(tok, count):
                v = idx_ref[tok]
                valid = (v >= 0) & (v < V_SHARD)

                @pl.when(valid)
                def _():
                    pltpu.make_async_copy(
                        e_hbm.at[pl.ds(v, 1), :],
                        out_ref.at[pl.ds(tok, 1), :],
                        gather_sems.at[k],
                    ).start()

                return count + valid.astype(jnp.int32)

            lo = c * chunk
            return lax.fori_loop(lo, lo + chunk, gather_one, jnp.int32(0))

        def drain_chunk(k, count):
            def drain_one(_, carry):
                pltpu.make_async_copy(
                    e_hbm.at[pl.ds(0, 1), :],
                    out_ref.at[pl.ds(0, 1), :],
                    gather_sems.at[k],
                ).wait()
                return carry

            lax.fori_loop(0, count, drain_one, jnp.int32(0))

        def rs_rdmas(s):
            c_cw = (my_y - s) % N_Y
            c_ccw = (my_y + s) % N_Y
            cw = pltpu.make_async_remote_copy(
                src_ref=out_ref.at[pl.ds(c_cw * chunk, chunk), pl.ds(0, dh)],
                dst_ref=tmp_ref.at[s, 0],
                send_sem=send_sems.at[s, 0],
                recv_sem=recv_sems.at[s, 0],
                device_id=(my_x, right, my_z),
                device_id_type=pl.DeviceIdType.MESH,
            )
            ccw = pltpu.make_async_remote_copy(
                src_ref=out_ref.at[pl.ds(c_ccw * chunk, chunk), pl.ds(dh, dh)],
                dst_ref=tmp_ref.at[s, 1],
                send_sem=send_sems.at[s, 1],
                recv_sem=recv_sems.at[s, 1],
                device_id=(my_x, left, my_z),
                device_id_type=pl.DeviceIdType.MESH,
            )
            return cw, ccw

        def rs_accumulate(s):
            r_cw = (my_y - s - 1) % N_Y
            r_ccw = (my_y + s + 1) % N_Y
            out_ref[pl.ds(r_cw * chunk, chunk), pl.ds(0, dh)] += tmp_ref[s, 0]
            out_ref[pl.ds(r_ccw * chunk, chunk), pl.ds(dh, dh)] += tmp_ref[s, 1]

        def ag_rdmas(g):
            s = (N_Y - 1) + g
            c_cw = (my_y + 1 - g) % N_Y
            c_ccw = (my_y - 1 + g) % N_Y
            cw = pltpu.make_async_remote_copy(
                src_ref=out_ref.at[pl.ds(c_cw * chunk, chunk), pl.ds(0, dh)],
                dst_ref=out_ref.at[pl.ds(c_cw * chunk, chunk), pl.ds(0, dh)],
                send_sem=send_sems.at[s, 0],
                recv_sem=recv_sems.at[s, 0],
                device_id=(my_x, right, my_z),
                device_id_type=pl.DeviceIdType.MESH,
            )
            ccw = pltpu.make_async_remote_copy(
                src_ref=out_ref.at[pl.ds(c_ccw * chunk, chunk), pl.ds(dh, dh)],
                dst_ref=out_ref.at[pl.ds(c_ccw * chunk, chunk), pl.ds(dh, dh)],
                send_sem=send_sems.at[s, 1],
                recv_sem=recv_sems.at[s, 1],
                device_id=(my_x, left, my_z),
                device_id_type=pl.DeviceIdType.MESH,
            )
            return cw, ccw

        if not _SKIP_GATHER:
            cnt0 = scan_chunk(my_y, 0)
            drain_chunk(0, cnt0)

        if _SKIP_RING:
            return

        pl.semaphore_wait(barrier_sem, 2)
        rs0_cw, rs0_ccw = rs_rdmas(0)
        rs0_cw.start()
        rs0_ccw.start()

        if not _SKIP_GATHER:
            cnt1 = scan_chunk(left, 1)
            drain_chunk(1, cnt1)
            cnt2 = scan_chunk(right, 2)
            drain_chunk(2, cnt2)

        rs0_cw.wait()
        rs0_ccw.wait()
        rs_accumulate(0)

        rs1_cw, rs1_ccw = rs_rdmas(1)
        rs1_cw.start()
        rs1_ccw.start()

        if not _SKIP_GATHER:
            cnt3 = scan_chunk((my_y + 2) % N_Y, 3)
            drain_chunk(3, cnt3)

        rs1_cw.wait()
        rs1_ccw.wait()
        rs_accumulate(1)

        rs2_cw, rs2_ccw = rs_rdmas(2)
        rs2_cw.start()
        rs2_ccw.start()
        rs2_cw.wait()
        rs2_ccw.wait()
        rs_accumulate(2)

        for g in range(N_Y - 1):
            ag_cw, ag_ccw = ag_rdmas(g)
            ag_cw.start()
            ag_ccw.start()
            ag_cw.wait()
            ag_ccw.wait()

    return pl.pallas_call(
        body,
        out_shape=jax.ShapeDtypeStruct((t, d), E.dtype),
        in_specs=[
            pl.BlockSpec(memory_space=pltpu.SMEM),
            pl.BlockSpec(memory_space=pl.ANY),
        ],
        out_specs=pl.BlockSpec(memory_space=pltpu.VMEM),
        scratch_shapes=[
            pltpu.VMEM((N_Y - 1, 2, chunk, dh), E.dtype),
            pltpu.SemaphoreType.DMA((n_steps, 2)),
            pltpu.SemaphoreType.DMA((n_steps, 2)),
            pltpu.SemaphoreType.DMA((N_Y,)),
        ],
        compiler_params=pltpu.CompilerParams(
            collective_id=0,
            vmem_limit_bytes=60 * 1024 * 1024,
        ),
    )(local, E)
